# Initial kernel scaffold; baseline (speedup 1.0000x reference)
import jax
import jax.numpy as jnp
from jax.experimental import pallas as pl


def kernel(bpreds, blls, bseqs, ti):
    raise NotImplementedError("write your pallas kernel here")


# speedup vs baseline: 1.2034x; 1.2034x over previous
import functools

import jax
import jax.numpy as jnp
from jax.experimental import pallas as pl
from jax.experimental.pallas import tpu as pltpu

BEAMS = 8
NEG = -3.0e38


def _step_kernel(ti_ref, preds_ref, blls_ref, seqs_ref,
                 blls_out_ref, nt_out_ref, seqs_out_ref):
    x = preds_ref[0]
    bll = blls_ref[0]
    seqs = seqs_ref[0]
    V = x.shape[1]

    col = jax.lax.broadcasted_iota(jnp.int32, x.shape, 1)

    m = jnp.max(x, axis=1, keepdims=True)
    s = jnp.sum(jnp.exp(x - m), axis=1, keepdims=True)
    lse = m + jnp.log(s)

    vals = []
    idxs = []
    x_cur = x
    for _ in range(BEAMS):
        v = jnp.max(x_cur, axis=1, keepdims=True)
        eq = x_cur == v
        idx = jnp.min(jnp.where(eq, col, V), axis=1, keepdims=True)
        x_cur = jnp.where(col == idx, NEG, x_cur)
        vals.append(v)
        idxs.append(idx)
    topv = jnp.concatenate(vals, axis=1)
    BC = jnp.concatenate(idxs, axis=1)

    A = jnp.log(jnp.exp(topv - lse) + 1e-8) + bll

    F = (jax.lax.broadcasted_iota(jnp.int32, (BEAMS, BEAMS), 0) * BEAMS
         + jax.lax.broadcasted_iota(jnp.int32, (BEAMS, BEAMS), 1))
    R = jnp.zeros((BEAMS, BEAMS), jnp.int32)
    for i2 in range(BEAMS):
        for k2 in range(BEAMS):
            a = A[i2, k2]
            f = i2 * BEAMS + k2
            R = R + jnp.where((a > A) | ((a == A) & (f < F)), 1, 0)

    lane8 = jax.lax.broadcasted_iota(jnp.int32, (1, BEAMS), 1)
    row8 = jax.lax.broadcasted_iota(jnp.int32, (BEAMS, 1), 0)
    seq_col = jax.lax.broadcasted_iota(jnp.int32, (1, seqs.shape[1]), 1)
    t_pos = ti_ref[0] + 1

    new_blls = jnp.zeros((1, BEAMS), jnp.float32)
    new_nt = jnp.zeros((1, BEAMS), jnp.int32)
    for k in range(BEAMS):
        sel = R == k
        e_ll = jnp.sum(jnp.where(sel, A, 0.0))
        ntk = jnp.sum(jnp.where(sel, BC, 0))
        old = jnp.sum(jnp.where(sel, row8, 0))
        new_blls = jnp.where(lane8 == k, e_ll, new_blls)
        new_nt = jnp.where(lane8 == k, ntk, new_nt)
        picked = jnp.sum(jnp.where(row8 == old, seqs, 0),
                         axis=0, keepdims=True)
        seqs_out_ref[0, k, :] = jnp.where(seq_col == t_pos, ntk, picked)[0]

    blls_out_ref[0] = new_blls
    nt_out_ref[0] = new_nt


@jax.jit
def kernel(bpreds, blls, bseqs, ti):
    BT, V = bpreds.shape
    B = BT // BEAMS
    SEQ = bseqs.shape[1]
    preds = bpreds.reshape(B, BEAMS, V)
    blls3 = blls.reshape(B, BEAMS, 1)
    seqs3 = bseqs.astype(jnp.int32).reshape(B, BEAMS, SEQ)
    ti_arr = jnp.full((1,), ti, jnp.int32)

    grid = (B,)
    out = pl.pallas_call(
        _step_kernel,
        grid=grid,
        in_specs=[
            pl.BlockSpec(memory_space=pltpu.SMEM),
            pl.BlockSpec((1, BEAMS, V), lambda b: (b, 0, 0)),
            pl.BlockSpec((1, BEAMS, 1), lambda b: (b, 0, 0)),
            pl.BlockSpec((1, BEAMS, SEQ), lambda b: (b, 0, 0)),
        ],
        out_specs=[
            pl.BlockSpec((1, 1, BEAMS), lambda b: (b, 0, 0)),
            pl.BlockSpec((1, 1, BEAMS), lambda b: (b, 0, 0)),
            pl.BlockSpec((1, BEAMS, SEQ), lambda b: (b, 0, 0)),
        ],
        out_shape=[
            jax.ShapeDtypeStruct((B, 1, BEAMS), jnp.float32),
            jax.ShapeDtypeStruct((B, 1, BEAMS), jnp.int32),
            jax.ShapeDtypeStruct((B, BEAMS, SEQ), jnp.int32),
        ],
    )(ti_arr, preds, blls3, seqs3)

    new_blls = out[0].reshape(BT)
    nt = out[1].reshape(BT)
    new_bseqs = out[2].reshape(BT, SEQ)
    return (new_blls, nt, new_bseqs)
